# Initial kernel scaffold; baseline (speedup 1.0000x reference)
import jax
import jax.numpy as jnp
from jax.experimental import pallas as pl


def kernel(x, edge_index, W1_0, b1_0, W2_0, b2_0, W1_1, b1_1, W2_1, b2_1, W1_2, b1_2, W2_2, b2_2, lin_W, lin_b):
    raise NotImplementedError("write your pallas kernel here")


# speedup vs baseline: 3.4384x; 3.4384x over previous
import functools

import jax
import jax.numpy as jnp
from jax import lax
from jax.experimental import pallas as pl
from jax.experimental.pallas import tpu as pltpu
from jax.experimental.pallas import tpu_sc as plsc

N = 10000
E = 160000
LB = 128
NS = 16
EPT = E // NS
NFULL = EPT // LB
REM = EPT % LB
RPT = 624
TAIL = N - NS * RPT

_f32 = jnp.float32


def _make_seg_sum(cb_total):
    my = cb_total // 2
    mesh = plsc.VectorSubcoreMesh(core_axis_name="c", subcore_axis_name="s")

    def body(*refs):
        src, dst, zs = refs[0:3]
        h_refs = refs[3:3 + cb_total]
        out_refs = refs[3 + cb_total:3 + 2 * cb_total]
        (shared, zbuf, sidx, didx, rows, sidx_r, didx_r, rows_r, sem) = \
            refs[3 + 2 * cb_total:]
        c = lax.axis_index("c")
        s = lax.axis_index("s")
        pltpu.sync_copy(zs, zbuf)
        ebase = s * EPT
        rbase = s * RPT
        for cb in range(cb_total):
            @pl.when(c == cb // my)
            def _(cb=cb):
                for t in range(RPT // LB):
                    pltpu.sync_copy(zbuf, shared.at[pl.ds(rbase + t * LB, LB)])
                if RPT % LB:
                    pltpu.sync_copy(
                        zbuf.at[pl.ds(0, RPT % LB)],
                        shared.at[pl.ds(rbase + (RPT // LB) * LB, RPT % LB)])

                @pl.when(s == 0)
                def _():
                    pltpu.sync_copy(zbuf.at[pl.ds(0, TAIL)],
                                    shared.at[pl.ds(NS * RPT, TAIL)])
                plsc.subcore_barrier()

                def chunk(j, carry):
                    off = pl.multiple_of(ebase + j * LB, 8)
                    pltpu.sync_copy(src.at[pl.ds(off, LB)], sidx)
                    pltpu.sync_copy(dst.at[pl.ds(off, LB)], didx)
                    pltpu.async_copy(h_refs[cb].at[sidx], rows, sem).wait()
                    pltpu.sync_copy(rows, shared.at[didx], add=True)
                    return carry

                lax.fori_loop(0, NFULL, chunk, 0)
                if REM:
                    off = pl.multiple_of(ebase + NFULL * LB, 8)
                    pltpu.sync_copy(src.at[pl.ds(off, REM)], sidx_r)
                    pltpu.sync_copy(dst.at[pl.ds(off, REM)], didx_r)
                    pltpu.async_copy(h_refs[cb].at[sidx_r], rows_r, sem).wait()
                    pltpu.sync_copy(rows_r, shared.at[didx_r], add=True)
                plsc.subcore_barrier()
                pltpu.sync_copy(shared.at[pl.ds(rbase, RPT)],
                                out_refs[cb].at[pl.ds(rbase, RPT)])

                @pl.when(s == 0)
                def _():
                    pltpu.sync_copy(shared.at[pl.ds(NS * RPT, TAIL)],
                                    out_refs[cb].at[pl.ds(NS * RPT, TAIL)])

    out_type = tuple(jax.ShapeDtypeStruct((N, LB), _f32)
                     for _ in range(cb_total))
    scratch = [
        pltpu.VMEM_SHARED((N, LB), _f32),
        pltpu.VMEM((LB, LB), _f32),
        pltpu.VMEM((LB,), jnp.int32),
        pltpu.VMEM((LB,), jnp.int32),
        pltpu.VMEM((LB, LB), _f32),
        pltpu.VMEM((REM,), jnp.int32),
        pltpu.VMEM((REM,), jnp.int32),
        pltpu.VMEM((REM, LB), _f32),
        pltpu.SemaphoreType.DMA,
    ]
    return pl.kernel(body, out_type=out_type, mesh=mesh,
                     scratch_types=scratch)


_ROWS = 1000


def _mlp_layer(cbi):
    ci = cbi * LB

    def body(*refs):
        h_refs = refs[0:cbi]
        a_refs = refs[cbi:2 * cbi]
        w1, b1, w2, b2 = refs[2 * cbi:2 * cbi + 4]
        o_refs = refs[2 * cbi + 4:]
        z = jnp.concatenate(
            [h_refs[i][...] + a_refs[i][...] for i in range(cbi)], axis=1)
        t = jnp.maximum(
            jnp.dot(z, w1[...], preferred_element_type=_f32) + b1[...], 0.0)
        y = jnp.maximum(
            jnp.dot(t, w2[...], preferred_element_type=_f32) + b2[...], 0.0)
        for i in range(4):
            o_refs[i][...] = y[:, LB * i:LB * (i + 1)]

    blk = pl.BlockSpec((_ROWS, LB), lambda i: (i, 0))
    in_specs = (
        [blk] * cbi + [blk] * cbi +
        [pl.BlockSpec((ci, 512), lambda i: (0, 0)),
         pl.BlockSpec((1, 512), lambda i: (0, 0)),
         pl.BlockSpec((512, 512), lambda i: (0, 0)),
         pl.BlockSpec((1, 512), lambda i: (0, 0))])
    out_specs = [blk] * 4
    return pl.pallas_call(
        body,
        grid=(N // _ROWS,),
        in_specs=in_specs,
        out_specs=out_specs,
        out_shape=tuple(jax.ShapeDtypeStruct((N, LB), _f32) for _ in range(4)),
    )


def _mlp_final():
    def body(*refs):
        h_refs = refs[0:4]
        a_refs = refs[4:8]
        w1, b1, w2, b2, lw, lb_, o_ref = refs[8:]
        z = jnp.concatenate(
            [h_refs[i][...] + a_refs[i][...] for i in range(4)], axis=1)
        t = jnp.maximum(
            jnp.dot(z, w1[...], preferred_element_type=_f32) + b1[...], 0.0)
        y = jnp.maximum(
            jnp.dot(t, w2[...], preferred_element_type=_f32) + b2[...], 0.0)
        o_ref[...] = jnp.dot(y, lw[...], preferred_element_type=_f32) + lb_[...]

    blk = pl.BlockSpec((_ROWS, LB), lambda i: (i, 0))
    in_specs = (
        [blk] * 8 +
        [pl.BlockSpec((512, 512), lambda i: (0, 0)),
         pl.BlockSpec((1, 512), lambda i: (0, 0)),
         pl.BlockSpec((512, 512), lambda i: (0, 0)),
         pl.BlockSpec((1, 512), lambda i: (0, 0)),
         pl.BlockSpec((512, 256), lambda i: (0, 0)),
         pl.BlockSpec((1, 256), lambda i: (0, 0))])
    return pl.pallas_call(
        body,
        grid=(N // _ROWS,),
        in_specs=in_specs,
        out_specs=pl.BlockSpec((_ROWS, 256), lambda i: (i, 0)),
        out_shape=jax.ShapeDtypeStruct((N, 256), _f32),
    )


def kernel(x, edge_index, W1_0, b1_0, W2_0, b2_0, W1_1, b1_1, W2_1, b2_1,
           W1_2, b1_2, W2_2, b2_2, lin_W, lin_b):
    src = edge_index[0]
    dst = edge_index[1]
    zeros128 = jnp.zeros((LB, LB), _f32)

    seg2 = _make_seg_sum(2)
    seg4 = _make_seg_sum(4)
    mlp2 = _mlp_layer(2)
    mlp4 = _mlp_layer(4)
    mlpf = _mlp_final()

    h = [x[:, :LB], x[:, LB:]]
    agg = seg2(src, dst, zeros128, *h)
    h = mlp2(*h, *agg, W1_0, b1_0.reshape(1, -1), W2_0, b2_0.reshape(1, -1))
    agg = seg4(src, dst, zeros128, *h)
    h = mlp4(*h, *agg, W1_1, b1_1.reshape(1, -1), W2_1, b2_1.reshape(1, -1))
    agg = seg4(src, dst, zeros128, *h)
    return mlpf(*h, *agg, W1_2, b1_2.reshape(1, -1), W2_2, b2_2.reshape(1, -1),
                lin_W, lin_b.reshape(1, -1))
